# baseline (speedup 1.0000x reference)
import functools

import jax
import jax.numpy as jnp
import numpy as np
from jax import lax
from jax.experimental import pallas as pl
from jax.experimental.pallas import tpu as pltpu
from jax.experimental.pallas import tpu_sc as plsc

_LANES = 16
_NW = 32

_KS0 = np.int32(0)
_KS1 = np.int32(42)
_KS2 = np.int32(np.uint32(0x1BD11BDA ^ 42).view(np.int32))
_ROTS_A = (13, 15, 26, 6)
_ROTS_B = (17, 29, 16, 24)
_THRESH = np.int32(419431)


def _drop_bits(j):
    x0 = jnp.zeros_like(j)
    x1 = j + _KS1
    inj = ((_KS1, _KS2, 1), (_KS2, _KS0, 2), (_KS0, _KS1, 3),
           (_KS1, _KS2, 4), (_KS2, _KS0, 5))
    for i, (ka, kb, cnt) in enumerate(inj):
        for r in (_ROTS_A if i % 2 == 0 else _ROTS_B):
            x0 = x0 + x1
            x1 = (x1 << r) | lax.shift_right_logical(x1, 32 - r)
            x1 = x1 ^ x0
        x0 = x0 + ka
        x1 = x1 + jnp.int32(kb + np.int32(cnt))
    return x0 ^ x1


def _make_drop(pad):
    chunk = pad // _NW
    vecs = chunk // _LANES
    mesh = plsc.VectorSubcoreMesh(core_axis_name="c", subcore_axis_name="s")

    @functools.partial(
        pl.kernel,
        mesh=mesh,
        out_type=[jax.ShapeDtypeStruct((pad,), jnp.int32),
                  jax.ShapeDtypeStruct((pad,), jnp.int32)],
        scratch_types=[pltpu.VMEM((chunk,), jnp.int32),
                       pltpu.VMEM((chunk,), jnp.int32)],
    )
    def drop_kernel(tr_hbm, te_hbm, tr_out, te_out, tr_v, te_v):
        wid = lax.axis_index("s") * 2 + lax.axis_index("c")
        base = wid * chunk
        pltpu.sync_copy(tr_hbm.at[pl.ds(base, chunk)], tr_v)
        pltpu.sync_copy(te_hbm.at[pl.ds(base, chunk)], te_v)
        lane = lax.iota(jnp.int32, _LANES)
        for v in range(vecs):
            j = base + jnp.int32(v * _LANES) + lane
            keep = lax.shift_right_logical(_drop_bits(j), 9) >= _THRESH
            sl = pl.ds(v * _LANES, _LANES)
            tr_v[sl] = jnp.where(keep, tr_v[sl], 0)
            te_v[sl] = jnp.where(keep, te_v[sl], 0)
        pltpu.sync_copy(tr_v, tr_out.at[pl.ds(base, chunk)])
        pltpu.sync_copy(te_v, te_out.at[pl.ds(base, chunk)])

    return drop_kernel


def kernel(x, y, train_mask, test_mask, edge_index):
    n = train_mask.shape[0]
    pad = -(-n // (_NW * _LANES)) * (_NW * _LANES)
    tr = jnp.pad(train_mask.astype(jnp.int32), (0, pad - n))
    te = jnp.pad(test_mask.astype(jnp.int32), (0, pad - n))
    tr_o, te_o = _make_drop(pad)(tr, te)
    return (x, edge_index, y,
            tr_o[:n].astype(bool), te_o[:n].astype(bool))


# speedup vs baseline: 1.0562x; 1.0562x over previous
import functools

import jax
import jax.numpy as jnp
import numpy as np
from jax import lax
from jax.experimental import pallas as pl
from jax.experimental.pallas import tpu as pltpu
from jax.experimental.pallas import tpu_sc as plsc

_LANES = 16
_NW = 32

_KS0 = np.int32(0)
_KS1 = np.int32(42)
_KS2 = np.int32(np.uint32(0x1BD11BDA ^ 42).view(np.int32))
_ROTS_A = (13, 15, 26, 6)
_ROTS_B = (17, 29, 16, 24)
_THRESH = np.int32(419431)


def _drop_bits(j):
    x0 = jnp.zeros_like(j)
    x1 = j + _KS1
    inj = ((_KS1, _KS2, 1), (_KS2, _KS0, 2), (_KS0, _KS1, 3),
           (_KS1, _KS2, 4), (_KS2, _KS0, 5))
    for i, (ka, kb, cnt) in enumerate(inj):
        for r in (_ROTS_A if i % 2 == 0 else _ROTS_B):
            x0 = x0 + x1
            x1 = (x1 << r) | lax.shift_right_logical(x1, 32 - r)
            x1 = x1 ^ x0
        x0 = x0 + ka
        x1 = x1 + jnp.int32(kb + np.int32(cnt))
    return x0 ^ x1


def _make_drop(pad):
    chunk = pad // _NW
    vecs = chunk // _LANES
    mesh = plsc.VectorSubcoreMesh(core_axis_name="c", subcore_axis_name="s")

    @functools.partial(
        pl.kernel,
        mesh=mesh,
        out_type=[jax.ShapeDtypeStruct((pad,), jnp.int32),
                  jax.ShapeDtypeStruct((pad,), jnp.int32)],
        scratch_types=[pltpu.VMEM((chunk,), jnp.int32),
                       pltpu.VMEM((chunk,), jnp.int32)],
    )
    def drop_kernel(tr_hbm, te_hbm, tr_out, te_out, tr_v, te_v):
        wid = lax.axis_index("s") * 2 + lax.axis_index("c")
        base = wid * chunk
        pltpu.sync_copy(tr_hbm.at[pl.ds(base, chunk)], tr_v)
        pltpu.sync_copy(te_hbm.at[pl.ds(base, chunk)], te_v)
        lane = lax.iota(jnp.int32, _LANES)

        def body(v, carry):
            j = base + v * _LANES + lane
            keep = lax.shift_right_logical(_drop_bits(j), 9) >= _THRESH
            sl = pl.ds(v * _LANES, _LANES)
            tr_v[sl] = jnp.where(keep, tr_v[sl], 0)
            te_v[sl] = jnp.where(keep, te_v[sl], 0)
            return carry

        lax.fori_loop(0, vecs, body, 0, unroll=False)
        pltpu.sync_copy(tr_v, tr_out.at[pl.ds(base, chunk)])
        pltpu.sync_copy(te_v, te_out.at[pl.ds(base, chunk)])

    return drop_kernel


def kernel(x, y, train_mask, test_mask, edge_index):
    n = train_mask.shape[0]
    pad = -(-n // (_NW * _LANES)) * (_NW * _LANES)
    tr = jnp.pad(train_mask.astype(jnp.int32), (0, pad - n))
    te = jnp.pad(test_mask.astype(jnp.int32), (0, pad - n))
    tr_o, te_o = _make_drop(pad)(tr, te)
    return (x, edge_index, y,
            tr_o[:n].astype(bool), te_o[:n].astype(bool))


# speedup vs baseline: 1.9231x; 1.8207x over previous
import functools

import jax
import jax.numpy as jnp
import numpy as np
from jax import lax
from jax.experimental import pallas as pl

_KS0 = np.int32(0)
_KS1 = np.int32(42)
_KS2 = np.int32(np.uint32(0x1BD11BDA ^ 42).view(np.int32))
_ROTS_A = (13, 15, 26, 6)
_ROTS_B = (17, 29, 16, 24)
_THRESH = np.int32(419431)


def _keep_bits(j):
    x0 = jnp.zeros_like(j)
    x1 = j + _KS1
    inj = ((_KS1, _KS2, 1), (_KS2, _KS0, 2), (_KS0, _KS1, 3),
           (_KS1, _KS2, 4), (_KS2, _KS0, 5))
    for i, (ka, kb, cnt) in enumerate(inj):
        for r in (_ROTS_A if i % 2 == 0 else _ROTS_B):
            x0 = x0 + x1
            x1 = (x1 << r) | lax.shift_right_logical(x1, 32 - r)
            x1 = x1 ^ x0
        x0 = x0 + ka
        x1 = x1 + jnp.int32(kb + np.int32(cnt))
    return lax.shift_right_logical(x0 ^ x1, 9) >= _THRESH


def _body(x_in, e_in, y_in, tr_in, te_in,
          x_out, e_out, y_out, tr_out, te_out):
    step = pl.program_id(0)
    x_out[...] = x_in[...]
    e_out[...] = e_in[...]

    @pl.when(step == 0)
    def _():
        y_out[...] = y_in[...]
        n = tr_in.shape[0]
        keep = _keep_bits(lax.broadcasted_iota(jnp.int32, (n,), 0))
        tr_out[...] = keep & tr_in[...]
        te_out[...] = keep & te_in[...]


def _make(n, d, e, steps):
    grid = (steps,)
    return pl.pallas_call(
        _body,
        grid=grid,
        in_specs=[
            pl.BlockSpec((n // steps, d), lambda i: (i, 0)),
            pl.BlockSpec((2, e // steps), lambda i: (0, i)),
            pl.BlockSpec((n,), lambda i: (0,)),
            pl.BlockSpec((n,), lambda i: (0,)),
            pl.BlockSpec((n,), lambda i: (0,)),
        ],
        out_specs=[
            pl.BlockSpec((n // steps, d), lambda i: (i, 0)),
            pl.BlockSpec((2, e // steps), lambda i: (0, i)),
            pl.BlockSpec((n,), lambda i: (0,)),
            pl.BlockSpec((n,), lambda i: (0,)),
            pl.BlockSpec((n,), lambda i: (0,)),
        ],
        out_shape=[
            jax.ShapeDtypeStruct((n, d), jnp.float32),
            jax.ShapeDtypeStruct((2, e), jnp.int32),
            jax.ShapeDtypeStruct((n,), jnp.int32),
            jax.ShapeDtypeStruct((n,), jnp.bool_),
            jax.ShapeDtypeStruct((n,), jnp.bool_),
        ],
    )


def kernel(x, y, train_mask, test_mask, edge_index):
    n, d = x.shape
    e = edge_index.shape[1]
    x_o, e_o, y_o, tr_o, te_o = _make(n, d, e, 10)(
        x, edge_index, y, train_mask, test_mask)
    return (x_o, e_o, y_o, tr_o, te_o)
